# baseline (speedup 1.0000x reference)
import functools

import jax
import jax.numpy as jnp
from jax import lax
from jax.experimental import pallas as pl
from jax.experimental.pallas import tpu as pltpu
from jax.experimental.pallas import tpu_sc as plsc

N = 320000
D = 128
S = 10000

NC, NS, L = 2, 16, 16
NW = NC * NS
R = N // NW
VPT = R // L
SPAD = 10240
SEG_C = SPAD // NW
SEG_V = SEG_C // L
_UNROLL = 5

_GRP = 100
_NROW = N // 128
_mesh = plsc.VectorSubcoreMesh(
    core_axis_name="c", subcore_axis_name="s", num_cores=NC, num_subcores=NS
)
_params = pltpu.CompilerParams(needs_layout_passes=False)


def _wid():
    return lax.axis_index("s") * NC + lax.axis_index("c")


def _latent_body(x_ref, w_ref, b_ref, o_ref):
    x3 = x_ref[...].reshape(_GRP, 128, D)
    lat = lax.dot_general(
        w_ref[...], x3, (((1,), (2,)), ((), ())),
        preferred_element_type=jnp.float32,
    ).reshape(1, _GRP, 128) + b_ref[0, 0]
    o_ref[...] = jnp.where(lat >= 0, lat, 0.2 * lat)


def _latent(x, W, b):
    nblk = _NROW // _GRP
    return pl.pallas_call(
        _latent_body,
        grid=(nblk,),
        in_specs=[
            pl.BlockSpec((1, _GRP, 128, D), lambda i: (i, 0, 0, 0)),
            pl.BlockSpec((1, D), lambda i: (0, 0)),
            pl.BlockSpec(memory_space=pltpu.SMEM),
        ],
        out_specs=pl.BlockSpec((1, _GRP, 128), lambda i: (i, 0, 0)),
        out_shape=jax.ShapeDtypeStruct((nblk, _GRP, 128), jnp.float32),
    )(x.reshape(nblk, _GRP, 128, D), W, b.reshape(1, 1))


_II = None


def _lanes():
    ii = lax.iota(jnp.int32, L)
    return ii


_GDN = lax.GatherDimensionNumbers(
    offset_dims=(), collapsed_slice_dims=(0,), start_index_map=(0,)
)


def _shuffle(vec, lanes_src):
    return lax.gather(
        vec, lanes_src[:, None], _GDN, (1,),
        mode=lax.GatherScatterMode.PROMISE_IN_BOUNDS,
    )


def _run_masks(idx_vec, ii):
    prev = _shuffle(idx_vec, jnp.maximum(ii - 1, 0))
    is_start = jnp.logical_or(ii == 0, idx_vec != prev)
    nxt = _shuffle(idx_vec, jnp.minimum(ii + 1, L - 1))
    is_last = jnp.logical_or(ii == L - 1, idx_vec != nxt)
    return is_start, is_last


def _k1_body(lat_hbm, idx_hbm, lmax_hbm, lat_v, idx_v, acc_v):
    wid = _wid()
    base = wid * R
    pltpu.sync_copy(lat_hbm.at[pl.ds(base, R)], lat_v)
    pltpu.sync_copy(idx_hbm.at[pl.ds(base, R)], idx_v)
    ii = _lanes()
    neg = jnp.full((L,), -jnp.inf, jnp.float32)

    def init(v, _):
        acc_v[pl.ds(v * L, L)] = neg
        return _
    lax.fori_loop(0, SPAD // L, init, None)

    l15 = jnp.full((L,), L - 1, jnp.int32)

    def step(v, carry):
        jprev, cprev = carry
        for u in range(_UNROLL):
            o = (v * _UNROLL + u) * L
            j = idx_v[pl.ds(o, L)]
            x = lat_v[pl.ds(o, L)]
            fold = jnp.logical_and(ii == 0, j == jprev)
            c = jnp.where(fold, jnp.maximum(x, cprev), x)
            for s in (1, 2, 4, 8):
                lanes = jnp.maximum(ii - s, 0)
                cs = _shuffle(c, lanes)
                js = _shuffle(j, lanes)
                c = jnp.where(js == j, jnp.maximum(c, cs), c)
            tmask = jnp.logical_and(
                jnp.logical_and(ii == 0, j != jprev), jprev >= 0)
            plsc.store_scatter(acc_v, [jprev], cprev, mask=tmask)
            nxt = _shuffle(j, jnp.minimum(ii + 1, L - 1))
            is_last = j != nxt
            plsc.store_scatter(acc_v, [j], c, mask=is_last)
            jprev = _shuffle(j, l15)
            cprev = _shuffle(c, l15)
        return jprev, cprev
    jprev, cprev = lax.fori_loop(
        0, VPT // _UNROLL, step, (jnp.full((L,), -1, jnp.int32), neg))
    plsc.store_scatter(acc_v, [jprev], cprev, mask=ii == 0)
    pltpu.sync_copy(acc_v, lmax_hbm.at[pl.ds(wid * SPAD, SPAD)])


_k1 = pl.kernel(
    _k1_body,
    out_type=jax.ShapeDtypeStruct((NW * SPAD,), jnp.float32),
    mesh=_mesh,
    compiler_params=_params,
    scratch_types=[
        pltpu.VMEM((R,), jnp.float32),
        pltpu.VMEM((R,), jnp.int32),
        pltpu.VMEM((SPAD,), jnp.float32),
    ],
)


def _combine_body(part_hbm, out_hbm, part_v, acc_v, sem, *, is_max):
    wid = _wid()
    off = wid * SEG_C
    copies = [
        pltpu.make_async_copy(part_hbm.at[pl.ds(r * SPAD + off, SEG_C)],
                              part_v.at[pl.ds(r * SEG_C, SEG_C)], sem)
        for r in range(NW)
    ]
    for c in copies:
        c.start()
    for c in copies:
        c.wait()
    for v in range(SEG_V):
        a = part_v[pl.ds(v * L, L)]
        for r in range(1, NW):
            b = part_v[pl.ds(r * SEG_C + v * L, L)]
            a = jnp.maximum(a, b) if is_max else a + b
        acc_v[pl.ds(v * L, L)] = a if is_max else 1.0 / a
    pltpu.sync_copy(acc_v, out_hbm.at[pl.ds(off, SEG_C)])


def _make_combine(is_max):
    return pl.kernel(
        functools.partial(_combine_body, is_max=is_max),
        out_type=jax.ShapeDtypeStruct((SPAD,), jnp.float32),
        mesh=_mesh,
        compiler_params=_params,
        scratch_types=[
            pltpu.VMEM((NW * SEG_C,), jnp.float32),
            pltpu.VMEM((SEG_C,), jnp.float32),
            pltpu.SemaphoreType.DMA,
        ],
    )


_k2 = _make_combine(True)
_k4 = _make_combine(False)


def _k3_body(lat_hbm, idx_hbm, gmax_hbm, e_hbm, lsum_hbm,
             lat_v, idx_v, gmax_v, acc_v, e_v):
    wid = _wid()
    base = wid * R
    pltpu.sync_copy(lat_hbm.at[pl.ds(base, R)], lat_v)
    pltpu.sync_copy(idx_hbm.at[pl.ds(base, R)], idx_v)
    pltpu.sync_copy(gmax_hbm, gmax_v)
    ii = _lanes()
    zero = jnp.zeros((L,), jnp.float32)

    def init(v, _):
        acc_v[pl.ds(v * L, L)] = zero
        return _
    lax.fori_loop(0, SPAD // L, init, None)

    l15 = jnp.full((L,), L - 1, jnp.int32)

    def step(v, carry):
        jprev, cprev = carry
        for u in range(_UNROLL):
            o = (v * _UNROLL + u) * L
            j = idx_v[pl.ds(o, L)]
            x = lat_v[pl.ds(o, L)]
            gm = plsc.load_gather(gmax_v, [j])
            e = jnp.exp(x - gm)
            e_v[pl.ds(o, L)] = e
            fold = jnp.logical_and(ii == 0, j == jprev)
            c = e + jnp.where(fold, cprev, zero)
            for s in (1, 2, 4, 8):
                lanes = jnp.maximum(ii - s, 0)
                cs = _shuffle(c, lanes)
                js = _shuffle(j, lanes)
                ok = jnp.logical_and(ii >= s, js == j)
                c = c + jnp.where(ok, cs, zero)
            tmask = jnp.logical_and(
                jnp.logical_and(ii == 0, j != jprev), jprev >= 0)
            plsc.store_scatter(acc_v, [jprev], cprev, mask=tmask)
            nxt = _shuffle(j, jnp.minimum(ii + 1, L - 1))
            is_last = j != nxt
            plsc.store_scatter(acc_v, [j], c, mask=is_last)
            jprev = _shuffle(j, l15)
            cprev = _shuffle(c, l15)
        return jprev, cprev
    jprev, cprev = lax.fori_loop(
        0, VPT // _UNROLL, step, (jnp.full((L,), -1, jnp.int32), zero))
    plsc.store_scatter(acc_v, [jprev], cprev, mask=ii == 0)
    pltpu.sync_copy(e_v, e_hbm.at[pl.ds(base, R)])
    pltpu.sync_copy(acc_v, lsum_hbm.at[pl.ds(wid * SPAD, SPAD)])


_k3 = pl.kernel(
    _k3_body,
    out_type=(
        jax.ShapeDtypeStruct((N,), jnp.float32),
        jax.ShapeDtypeStruct((NW * SPAD,), jnp.float32),
    ),
    mesh=_mesh,
    compiler_params=_params,
    scratch_types=[
        pltpu.VMEM((R,), jnp.float32),
        pltpu.VMEM((R,), jnp.int32),
        pltpu.VMEM((SPAD,), jnp.float32),
        pltpu.VMEM((SPAD,), jnp.float32),
        pltpu.VMEM((R,), jnp.float32),
    ],
)


def _k5_body(e_hbm, idx_hbm, gsum_hbm, out_hbm, e_v, idx_v, gsum_v, out_v):
    wid = _wid()
    base = wid * R
    pltpu.sync_copy(e_hbm.at[pl.ds(base, R)], e_v)
    pltpu.sync_copy(idx_hbm.at[pl.ds(base, R)], idx_v)
    pltpu.sync_copy(gsum_hbm, gsum_v)

    def step(v, _):
        for u in range(_UNROLL):
            o = (v * _UNROLL + u) * L
            j = idx_v[pl.ds(o, L)]
            gs = plsc.load_gather(gsum_v, [j])
            out_v[pl.ds(o, L)] = e_v[pl.ds(o, L)] * gs
        return _
    lax.fori_loop(0, VPT // _UNROLL, step, None)
    pltpu.sync_copy(out_v, out_hbm.at[pl.ds(base, R)])


_k5 = pl.kernel(
    _k5_body,
    out_type=jax.ShapeDtypeStruct((N,), jnp.float32),
    mesh=_mesh,
    compiler_params=_params,
    scratch_types=[
        pltpu.VMEM((R,), jnp.float32),
        pltpu.VMEM((R,), jnp.int32),
        pltpu.VMEM((SPAD,), jnp.float32),
        pltpu.VMEM((R,), jnp.float32),
    ],
)


@jax.jit
def kernel(x, W, b, index):
    lat = _latent(x, W, b).reshape(N)
    lmax = _k1(lat, index)
    gmax = _k2(lmax)
    e, lsum = _k3(lat, index, gmax)
    gsum = _k4(lsum)
    out = _k5(e, index, gsum)
    return out.reshape(N, 1)


# speedup vs baseline: 1.0998x; 1.0998x over previous
import functools

import jax
import jax.numpy as jnp
from jax import lax
from jax.experimental import pallas as pl
from jax.experimental.pallas import tpu as pltpu
from jax.experimental.pallas import tpu_sc as plsc

N = 320000
D = 128
S = 10000

NC, NS, L = 2, 16, 16
NW = NC * NS
R = N // NW
VPT = R // L
SPAD = 10240
SEG_C = SPAD // NW
SEG_V = SEG_C // L
_UNROLL = 5

_GRP = 100
_NROW = N // 128
_mesh = plsc.VectorSubcoreMesh(
    core_axis_name="c", subcore_axis_name="s", num_cores=NC, num_subcores=NS
)
_params = pltpu.CompilerParams(needs_layout_passes=False)


def _wid():
    return lax.axis_index("s") * NC + lax.axis_index("c")


def _latent_body(x_ref, w_ref, b_ref, o_ref):
    x3 = x_ref[...].reshape(_GRP, 128, D)
    lat = lax.dot_general(
        w_ref[...], x3, (((1,), (2,)), ((), ())),
        preferred_element_type=jnp.float32,
    ).reshape(1, _GRP, 128) + b_ref[0, 0]
    o_ref[...] = jnp.where(lat >= 0, lat, 0.2 * lat)


def _latent(x, W, b):
    nblk = _NROW // _GRP
    return pl.pallas_call(
        _latent_body,
        grid=(nblk,),
        in_specs=[
            pl.BlockSpec((1, _GRP, 128, D), lambda i: (i, 0, 0, 0)),
            pl.BlockSpec((1, D), lambda i: (0, 0)),
            pl.BlockSpec(memory_space=pltpu.SMEM),
        ],
        out_specs=pl.BlockSpec((1, _GRP, 128), lambda i: (i, 0, 0)),
        out_shape=jax.ShapeDtypeStruct((nblk, _GRP, 128), jnp.float32),
    )(x.reshape(nblk, _GRP, 128, D), W, b.reshape(1, 1))


_II = None


def _lanes():
    ii = lax.iota(jnp.int32, L)
    return ii


_GDN = lax.GatherDimensionNumbers(
    offset_dims=(), collapsed_slice_dims=(0,), start_index_map=(0,)
)


def _shuffle(vec, lanes_src):
    return lax.gather(
        vec, lanes_src[:, None], _GDN, (1,),
        mode=lax.GatherScatterMode.PROMISE_IN_BOUNDS,
    )


def _run_masks(idx_vec, ii):
    prev = _shuffle(idx_vec, jnp.maximum(ii - 1, 0))
    is_start = jnp.logical_or(ii == 0, idx_vec != prev)
    nxt = _shuffle(idx_vec, jnp.minimum(ii + 1, L - 1))
    is_last = jnp.logical_or(ii == L - 1, idx_vec != nxt)
    return is_start, is_last


def _k1_body(lat_hbm, idx_hbm, lmax_hbm, lat_v, idx_v, acc_v):
    wid = _wid()
    base = wid * R
    pltpu.sync_copy(lat_hbm.at[pl.ds(base, R)], lat_v)
    pltpu.sync_copy(idx_hbm.at[pl.ds(base, R)], idx_v)
    ii = _lanes()
    neg = jnp.full((L,), -jnp.inf, jnp.float32)

    def init(v, _):
        acc_v[pl.ds(v * L, L)] = neg
        return _
    lax.fori_loop(0, SPAD // L, init, None)

    l15 = jnp.full((L,), L - 1, jnp.int32)

    @plsc.parallel_loop(0, R, L, unroll=_UNROLL,
                        carry=(jnp.full((L,), -1, jnp.int32), neg))
    def step(o, carry):
        jprev, cprev = carry
        j = idx_v[pl.ds(o, L)]
        x = lat_v[pl.ds(o, L)]
        fold = jnp.logical_and(ii == 0, j == jprev)
        c = jnp.where(fold, jnp.maximum(x, cprev), x)
        for s in (1, 2, 4, 8):
            lanes = jnp.maximum(ii - s, 0)
            cs = _shuffle(c, lanes)
            js = _shuffle(j, lanes)
            c = jnp.where(js == j, jnp.maximum(c, cs), c)
        tmask = jnp.logical_and(
            jnp.logical_and(ii == 0, j != jprev), jprev >= 0)
        plsc.store_scatter(acc_v, [jprev], cprev, mask=tmask)
        nxt = _shuffle(j, jnp.minimum(ii + 1, L - 1))
        is_last = j != nxt
        plsc.store_scatter(acc_v, [j], c, mask=is_last)
        return _shuffle(j, l15), _shuffle(c, l15)
    jprev, cprev = step
    plsc.store_scatter(acc_v, [jprev], cprev, mask=ii == 0)
    pltpu.sync_copy(acc_v, lmax_hbm.at[pl.ds(wid * SPAD, SPAD)])


_k1 = pl.kernel(
    _k1_body,
    out_type=jax.ShapeDtypeStruct((NW * SPAD,), jnp.float32),
    mesh=_mesh,
    compiler_params=_params,
    scratch_types=[
        pltpu.VMEM((R,), jnp.float32),
        pltpu.VMEM((R,), jnp.int32),
        pltpu.VMEM((SPAD,), jnp.float32),
    ],
)


def _combine_body(part_hbm, out_hbm, part_v, acc_v, sem, *, is_max):
    wid = _wid()
    off = wid * SEG_C
    copies = [
        pltpu.make_async_copy(part_hbm.at[pl.ds(r * SPAD + off, SEG_C)],
                              part_v.at[pl.ds(r * SEG_C, SEG_C)], sem)
        for r in range(NW)
    ]
    for c in copies:
        c.start()
    for c in copies:
        c.wait()
    for v in range(SEG_V):
        a = part_v[pl.ds(v * L, L)]
        for r in range(1, NW):
            b = part_v[pl.ds(r * SEG_C + v * L, L)]
            a = jnp.maximum(a, b) if is_max else a + b
        acc_v[pl.ds(v * L, L)] = a if is_max else 1.0 / a
    pltpu.sync_copy(acc_v, out_hbm.at[pl.ds(off, SEG_C)])


def _make_combine(is_max):
    return pl.kernel(
        functools.partial(_combine_body, is_max=is_max),
        out_type=jax.ShapeDtypeStruct((SPAD,), jnp.float32),
        mesh=_mesh,
        compiler_params=_params,
        scratch_types=[
            pltpu.VMEM((NW * SEG_C,), jnp.float32),
            pltpu.VMEM((SEG_C,), jnp.float32),
            pltpu.SemaphoreType.DMA,
        ],
    )


_k2 = _make_combine(True)
_k4 = _make_combine(False)


def _k3_body(lat_hbm, idx_hbm, gmax_hbm, e_hbm, lsum_hbm,
             lat_v, idx_v, gmax_v, acc_v, e_v):
    wid = _wid()
    base = wid * R
    pltpu.sync_copy(lat_hbm.at[pl.ds(base, R)], lat_v)
    pltpu.sync_copy(idx_hbm.at[pl.ds(base, R)], idx_v)
    pltpu.sync_copy(gmax_hbm, gmax_v)
    ii = _lanes()
    zero = jnp.zeros((L,), jnp.float32)

    def init(v, _):
        acc_v[pl.ds(v * L, L)] = zero
        return _
    lax.fori_loop(0, SPAD // L, init, None)

    l15 = jnp.full((L,), L - 1, jnp.int32)

    @plsc.parallel_loop(0, R, L, unroll=_UNROLL,
                        carry=(jnp.full((L,), -1, jnp.int32), zero))
    def step(o, carry):
        jprev, cprev = carry
        j = idx_v[pl.ds(o, L)]
        x = lat_v[pl.ds(o, L)]
        gm = plsc.load_gather(gmax_v, [j])
        e = jnp.exp(x - gm)
        e_v[pl.ds(o, L)] = e
        fold = jnp.logical_and(ii == 0, j == jprev)
        c = e + jnp.where(fold, cprev, zero)
        for s in (1, 2, 4, 8):
            lanes = jnp.maximum(ii - s, 0)
            cs = _shuffle(c, lanes)
            js = _shuffle(j, lanes)
            ok = jnp.logical_and(ii >= s, js == j)
            c = c + jnp.where(ok, cs, zero)
        tmask = jnp.logical_and(
            jnp.logical_and(ii == 0, j != jprev), jprev >= 0)
        plsc.store_scatter(acc_v, [jprev], cprev, mask=tmask)
        nxt = _shuffle(j, jnp.minimum(ii + 1, L - 1))
        is_last = j != nxt
        plsc.store_scatter(acc_v, [j], c, mask=is_last)
        return _shuffle(j, l15), _shuffle(c, l15)
    jprev, cprev = step
    plsc.store_scatter(acc_v, [jprev], cprev, mask=ii == 0)
    pltpu.sync_copy(e_v, e_hbm.at[pl.ds(base, R)])
    pltpu.sync_copy(acc_v, lsum_hbm.at[pl.ds(wid * SPAD, SPAD)])


_k3 = pl.kernel(
    _k3_body,
    out_type=(
        jax.ShapeDtypeStruct((N,), jnp.float32),
        jax.ShapeDtypeStruct((NW * SPAD,), jnp.float32),
    ),
    mesh=_mesh,
    compiler_params=_params,
    scratch_types=[
        pltpu.VMEM((R,), jnp.float32),
        pltpu.VMEM((R,), jnp.int32),
        pltpu.VMEM((SPAD,), jnp.float32),
        pltpu.VMEM((SPAD,), jnp.float32),
        pltpu.VMEM((R,), jnp.float32),
    ],
)


def _k5_body(e_hbm, idx_hbm, gsum_hbm, out_hbm, e_v, idx_v, gsum_v, out_v):
    wid = _wid()
    base = wid * R
    pltpu.sync_copy(e_hbm.at[pl.ds(base, R)], e_v)
    pltpu.sync_copy(idx_hbm.at[pl.ds(base, R)], idx_v)
    pltpu.sync_copy(gsum_hbm, gsum_v)

    @plsc.parallel_loop(0, R, L, unroll=_UNROLL)
    def step(o):
        j = idx_v[pl.ds(o, L)]
        gs = plsc.load_gather(gsum_v, [j])
        out_v[pl.ds(o, L)] = e_v[pl.ds(o, L)] * gs
    pltpu.sync_copy(out_v, out_hbm.at[pl.ds(base, R)])


_k5 = pl.kernel(
    _k5_body,
    out_type=jax.ShapeDtypeStruct((N,), jnp.float32),
    mesh=_mesh,
    compiler_params=_params,
    scratch_types=[
        pltpu.VMEM((R,), jnp.float32),
        pltpu.VMEM((R,), jnp.int32),
        pltpu.VMEM((SPAD,), jnp.float32),
        pltpu.VMEM((R,), jnp.float32),
    ],
)


@jax.jit
def kernel(x, W, b, index):
    lat = _latent(x, W, b).reshape(N)
    lmax = _k1(lat, index)
    gmax = _k2(lmax)
    e, lsum = _k3(lat, index, gmax)
    gsum = _k4(lsum)
    out = _k5(e, index, gsum)
    return out.reshape(N, 1)
